# Initial kernel scaffold; baseline (speedup 1.0000x reference)
import jax
import jax.numpy as jnp
from jax.experimental import pallas as pl


def kernel(X, Z, adj_e, adj_v, T, W1, b1, p1, W2, b2, p2, W3, b3, p3, W4, b4, p4, W5, b5, p5, W6, b6, p6, W7, b7, p7):
    raise NotImplementedError("write your pallas kernel here")


# speedup vs baseline: 1.2706x; 1.2706x over previous
import jax
import jax.numpy as jnp
from jax.experimental import pallas as pl
from jax.experimental.pallas import tpu as pltpu

_CFG = [
    (512, 128, 512, 512, True),
    (128, 128, 512, 128, False),
    (128, 32, 128, 128, True),
    (32, 32, 128, 32, False),
    (32, 4, 32, 32, True),
    (4, 4, 32, 4, False),
    (4, 1, 4, 4, True),
]


def _diag_one(mult):
    row = jax.lax.broadcasted_iota(jnp.int32, mult.shape, 0)
    col = jax.lax.broadcasted_iota(jnp.int32, mult.shape, 1)
    return jnp.where(row == col, jnp.float32(1.0), mult)


def _gcn_body(X_ref, Z_ref, adj_e_ref, adj_v_ref, T_ref, Tt_ref,
              W1, b1, p1, W2, b2, p2, W3, b3, p3, W4, b4, p4,
              W5, b5, p5, W6, b6, p6, W7, b7, p7, out_ref):
    Hv = X_ref[...]
    He = Z_ref[...]
    T = T_ref[...]
    Tt = Tt_ref[...]
    Av = adj_v_ref[...]
    Ae = adj_e_ref[...]

    Ws = (W1, W2, W3, W4, W5, W6, W7)
    bs = (b1, b2, b3, b4, b5, b6, b7)
    ps = (p1, p2, p3, p4, p5, p6, p7)

    nlayers = len(_CFG)
    for i, (iv, ov, ie, oe, node_layer) in enumerate(_CFG):
        W = Ws[i][...]
        b = bs[i][...]
        p = ps[i][...]
        if node_layer:
            d = jnp.dot(He, p, preferred_element_type=jnp.float32)
            mult = jnp.dot(T, Tt * d, preferred_element_type=jnp.float32)
            A = _diag_one(mult) * Av
            HW = jnp.dot(Hv, W, preferred_element_type=jnp.float32)
            Hv = jnp.dot(A, HW, preferred_element_type=jnp.float32) + b
        else:
            d = jnp.dot(Hv, p, preferred_element_type=jnp.float32)
            mult = jnp.dot(Tt, T * d, preferred_element_type=jnp.float32)
            A = _diag_one(mult) * Ae
            HW = jnp.dot(He, W, preferred_element_type=jnp.float32)
            He = jnp.dot(A, HW, preferred_element_type=jnp.float32) + b
        if i + 1 < nlayers:
            Hv = jnp.maximum(Hv, 0.0)
            He = jnp.maximum(He, 0.0)

    out_ref[...] = jax.nn.sigmoid(Hv)


def kernel(X, Z, adj_e, adj_v, T,
           W1, b1, p1, W2, b2, p2, W3, b3, p3, W4, b4, p4,
           W5, b5, p5, W6, b6, p6, W7, b7, p7):
    N = X.shape[0]
    Tt = T.T
    bs = [b1, b2, b3, b4, b5, b6, b7]
    ps = [p1, p2, p3, p4, p5, p6, p7]
    Ws = [W1, W2, W3, W4, W5, W6, W7]
    operands = [X, Z, adj_e, adj_v, T, Tt]
    for W, b, p in zip(Ws, bs, ps):
        operands += [W, b.reshape(1, -1), p.T]

    return pl.pallas_call(
        _gcn_body,
        out_shape=jax.ShapeDtypeStruct((N, 1), jnp.float32),
        compiler_params=pltpu.CompilerParams(
            vmem_limit_bytes=128 * 1024 * 1024,
        ),
    )(*operands)
